# baseline (speedup 1.0000x reference)
import functools

import jax
import jax.numpy as jnp
from jax import lax
from jax.experimental import pallas as pl
from jax.experimental.pallas import tpu as pltpu
from jax.experimental.pallas import tpu_sc as plsc

VOCAB = 100000
D = 128
P = 20
B = 1024
S = 220
T = S - P
HALF = T // 2

NC = 2
NS = 16
NW = NC * NS
BPW = B // NW
NPAIR = BPW // 2
SLOT = 2 * S
NBUF = 2

_GOFF = (P, P + HALF, S + P, S + P + HALF)

_MESH = plsc.VectorSubcoreMesh(
    core_axis_name="c", subcore_axis_name="s", num_cores=NC, num_subcores=NS
)


def _body(wte_hbm, ids_hbm, sp_hbm, out_hbm, idx_v, obuf, sem_g, sem_st):
    wid = lax.axis_index("s") * NC + lax.axis_index("c")

    pltpu.sync_copy(ids_hbm.at[wid], idx_v)
    for s in range(NBUF):
        pltpu.sync_copy(sp_hbm, obuf.at[s, pl.ds(0, P)])
        pltpu.sync_copy(sp_hbm, obuf.at[s, pl.ds(S, P)])

    def g_start(p, s):
        for h in range(4):
            pltpu.async_copy(
                wte_hbm.at[idx_v.at[p, h]],
                obuf.at[s, pl.ds(_GOFF[h], HALF)],
                sem_g.at[s],
            )

    def g_wait(s):
        for h in range(4):
            pltpu.make_async_copy(
                wte_hbm.at[idx_v.at[0, h]],
                obuf.at[s, pl.ds(_GOFF[h], HALF)],
                sem_g.at[s],
            ).wait()

    def st_start(p, s):
        pltpu.async_copy(
            obuf.at[s], out_hbm.at[pl.ds((wid * NPAIR + p) * SLOT, SLOT)],
            sem_st.at[s],
        )

    def st_wait(s):
        pltpu.make_async_copy(
            obuf.at[s], out_hbm.at[pl.ds(0, SLOT)], sem_st.at[s]
        ).wait()

    g_start(0, 0)
    g_start(1, 1)

    @pl.loop(0, NPAIR - NBUF, step=NBUF)
    def steady(i):
        for s in range(NBUF):
            g_wait(s)
            st_start(i + s, s)
            st_wait(s)
            g_start(i + s + NBUF, s)

    for s in range(NBUF):
        g_wait(s)
        st_start(NPAIR - NBUF + s, s)
        st_wait(s)


_sc_call = functools.partial(
    pl.kernel,
    out_type=jax.ShapeDtypeStruct((B * S, D), jnp.float32),
    mesh=_MESH,
    scratch_types=[
        pltpu.VMEM((NPAIR, 4, HALF), jnp.int32),
        pltpu.VMEM((NBUF, SLOT, D), jnp.float32),
        pltpu.SemaphoreType.DMA((NBUF,)),
        pltpu.SemaphoreType.DMA((NBUF,)),
    ],
)(_body)


@jax.jit
def kernel(input_ids, wte, softprompt):
    ids4 = input_ids[:, P:].reshape(NW, NPAIR, 4, HALF).astype(jnp.int32)
    return _sc_call(wte, ids4, softprompt).reshape(B, S, D)
